# baseline (speedup 1.0000x reference)
import functools

import jax
import jax.numpy as jnp
from jax import lax
from jax.experimental import pallas as pl
from jax.experimental.pallas import tpu as pltpu
from jax.experimental.pallas import tpu_sc as plsc

DIM = 64
NUM_CORES = 2
NUM_SUBCORES = 16
NUM_WORKERS = NUM_CORES * NUM_SUBCORES
CHUNK = 128


VOCAB = 1000000
N_TCHUNK = 7813
TAIL = N_TCHUNK - 1


def _transpose_kernel():
    mesh = plsc.VectorSubcoreMesh(
        core_axis_name="c",
        subcore_axis_name="s",
        num_cores=NUM_CORES,
        num_subcores=NUM_SUBCORES,
    )
    n_base, n_extra = divmod(N_TCHUNK, NUM_WORKERS)

    @functools.partial(
        pl.kernel,
        out_type=jax.ShapeDtypeStruct((VOCAB // 2, 2 * DIM), jnp.float32),
        mesh=mesh,
        scratch_types=[
            pltpu.VMEM((4, DIM, 2 * DIM), jnp.float32),
            pltpu.VMEM((4, DIM, 2 * DIM + 1), jnp.float32),
            pltpu.SemaphoreType.DMA,
            pltpu.SemaphoreType.DMA,
        ],
        compiler_params=pltpu.CompilerParams(
            use_tc_tiling_on_sc=True, needs_layout_passes=False
        ),
    )
    def kern(wt_hbm, pt_hbm, in_v, t_v, isem, osem):
        wid = lax.axis_index("s") * NUM_CORES + lax.axis_index("c")
        n_my = jnp.where(wid < n_extra, n_base + 1, n_base)

        def in_start(k, buf):
            c = wid + k * NUM_WORKERS
            pltpu.async_copy(
                wt_hbm.at[:, pl.ds(c * 128, 128)], in_v.at[buf], isem
            )

        def in_wait(k, buf):
            c = wid + k * NUM_WORKERS
            pltpu.make_async_copy(
                wt_hbm.at[:, pl.ds(c * 128, 128)], in_v.at[buf], isem
            ).wait()

        def out_start(k, buf):
            c = wid + k * NUM_WORKERS

            @pl.when(c != TAIL)
            def _():
                pltpu.async_copy(
                    t_v.at[buf, :, pl.ds(0, 2 * DIM)],
                    pt_hbm.at[pl.ds(c * 64, 64)],
                    osem,
                )

            @pl.when(c == TAIL)
            def _():
                pltpu.async_copy(
                    t_v.at[buf, pl.ds(0, 32), pl.ds(0, 2 * DIM)],
                    pt_hbm.at[pl.ds(c * 64, 32)],
                    osem,
                )

        def out_wait(k, buf):
            c = wid + k * NUM_WORKERS

            @pl.when(c != TAIL)
            def _():
                pltpu.make_async_copy(
                    t_v.at[buf, :, pl.ds(0, 2 * DIM)],
                    pt_hbm.at[pl.ds(c * 64, 64)],
                    osem,
                ).wait()

            @pl.when(c == TAIL)
            def _():
                pltpu.make_async_copy(
                    t_v.at[buf, pl.ds(0, 32), pl.ds(0, 2 * DIM)],
                    pt_hbm.at[pl.ds(c * 64, 32)],
                    osem,
                ).wait()

        lane = lax.iota(jnp.int32, 16)
        parity = (lane & 1) * DIM
        row_l0 = [(l0 + lane) >> 1 for l0 in range(0, 2 * DIM, 16)]

        for kk in range(3):
            @pl.when(kk < n_my)
            def _():
                in_start(kk, kk)

        def body(k, _):
            buf = lax.rem(k, 4)
            in_wait(k, buf)

            @pl.when(k + 3 < n_my)
            def _():
                in_start(k + 3, lax.rem(k + 3, 4))

            @pl.when(k >= 4)
            def _():
                out_wait(k - 4, buf)

            for f in range(DIM):
                cols = parity + f
                xs = [
                    in_v[buf, f, pl.ds(l0, 16)]
                    for l0 in range(0, 2 * DIM, 16)
                ]
                for li in range(len(xs)):
                    plsc.store_scatter(
                        t_v.at[buf], [row_l0[li], cols], xs[li]
                    )
            out_start(k, buf)
            return 0

        lax.fori_loop(0, n_my, body, 0, unroll=False)
        for kk in range(4):
            k_last = n_my - 4 + kk

            @pl.when(k_last >= 0)
            def _():
                out_wait(k_last, lax.rem(k_last, 4))

    return kern


def _gather_kernel(n_total):
    b_per_w = n_total // NUM_WORKERS
    n_chunks = b_per_w // CHUNK
    mesh = plsc.VectorSubcoreMesh(
        core_axis_name="c",
        subcore_axis_name="s",
        num_cores=NUM_CORES,
        num_subcores=NUM_SUBCORES,
    )

    @functools.partial(
        pl.kernel,
        out_type=jax.ShapeDtypeStruct((n_total, DIM), jnp.float32),
        mesh=mesh,
        scratch_types=[
            pltpu.VMEM((n_chunks, CHUNK), jnp.int32),
            pltpu.VMEM((2, CHUNK), jnp.int32),
            pltpu.VMEM((2, CHUNK, 2 * DIM), jnp.float32),
            pltpu.VMEM((2, CHUNK, DIM), jnp.float32),
            pltpu.SemaphoreType.DMA,
            pltpu.SemaphoreType.DMA,
        ],
        compiler_params=pltpu.CompilerParams(use_tc_tiling_on_sc=True),
    )
    def kern(idx_hbm, table_hbm, out_hbm, idx_v, pair_v, rows_v, sel_v, gsem, osem):
        wid = lax.axis_index("s") * NUM_CORES + lax.axis_index("c")
        base = wid * b_per_w
        pltpu.sync_copy(idx_hbm.at[wid], idx_v)

        def gather_start(c, buf):
            for v in range(CHUNK // 16):
                sl = pl.ds(v * 16, 16)
                pair_v[buf, sl] = jax.lax.shift_right_logical(idx_v[c, sl], 1)
            pltpu.async_copy(table_hbm.at[pair_v.at[buf]], rows_v.at[buf], gsem)

        def gather_wait(buf):
            pltpu.make_async_copy(
                table_hbm.at[pair_v.at[buf]], rows_v.at[buf], gsem
            ).wait()

        def out_start(c, buf):
            pltpu.async_copy(
                sel_v.at[buf], out_hbm.at[pl.ds(base + c * CHUNK, CHUNK)], osem
            )

        def out_wait(c, buf):
            pltpu.make_async_copy(
                sel_v.at[buf], out_hbm.at[pl.ds(base + c * CHUNK, CHUNK)], osem
            ).wait()

        gather_start(0, 0)

        def body(c, _):
            buf = lax.rem(c, 2)
            nxt = 1 - buf
            gather_wait(buf)

            @pl.when(c + 1 < n_chunks)
            def _():
                gather_start(c + 1, nxt)

            @pl.when(c >= 2)
            def _():
                out_wait(c - 2, buf)

            for r0 in range(0, CHUNK, 16):
                hv = (idx_v[c, pl.ds(r0, 16)] & 1) * DIM
                offs = [hv[j] for j in range(16)]
                for j0 in range(0, 16, 4):
                    xs = [
                        rows_v[buf, r0 + j0 + jj, pl.ds(offs[j0 + jj] + f0, 16)]
                        for jj in range(4)
                        for f0 in range(0, DIM, 16)
                    ]
                    for k, x in enumerate(xs):
                        jj, f0 = divmod(k, DIM // 16)
                        sel_v[buf, r0 + j0 + jj, pl.ds(f0 * 16, 16)] = x
            out_start(c, buf)
            return 0

        lax.fori_loop(0, n_chunks, body, 0, unroll=False)
        out_wait(n_chunks - 2, (n_chunks - 2) % 2)
        out_wait(n_chunks - 1, (n_chunks - 1) % 2)

    return kern


def kernel(source, W):
    n_total = source.shape[0] * source.shape[1]
    b_per_w = n_total // NUM_WORKERS
    table = _transpose_kernel()(W.T)
    idx = source.reshape(NUM_WORKERS, b_per_w // CHUNK, CHUNK).astype(jnp.int32)
    out = _gather_kernel(n_total)(idx, table)
    return out.reshape(source.shape[0], source.shape[1], DIM)


# speedup vs baseline: 1.5493x; 1.5493x over previous
import functools

import jax
import jax.numpy as jnp
from jax import lax
from jax.experimental import pallas as pl
from jax.experimental.pallas import tpu as pltpu
from jax.experimental.pallas import tpu_sc as plsc

DIM = 64
NUM_CORES = 2
NUM_SUBCORES = 16
NUM_WORKERS = NUM_CORES * NUM_SUBCORES
CHUNK = 128


def _gather_kernel(n_total):
    b_per_w = n_total // NUM_WORKERS
    n_chunks = b_per_w // CHUNK
    mesh = plsc.VectorSubcoreMesh(
        core_axis_name="c",
        subcore_axis_name="s",
        num_cores=NUM_CORES,
        num_subcores=NUM_SUBCORES,
    )

    @functools.partial(
        pl.kernel,
        out_type=jax.ShapeDtypeStruct((n_total, DIM), jnp.float32),
        mesh=mesh,
        scratch_types=[
            pltpu.VMEM((n_chunks, CHUNK), jnp.int32),
            pltpu.VMEM((2, CHUNK), jnp.int32),
            pltpu.VMEM((2, CHUNK, 2 * DIM), jnp.float32),
            pltpu.VMEM((2, CHUNK, DIM), jnp.float32),
            pltpu.SemaphoreType.DMA,
            pltpu.SemaphoreType.DMA,
        ],
        compiler_params=pltpu.CompilerParams(use_tc_tiling_on_sc=True),
    )
    def kern(idx_hbm, table_hbm, out_hbm, idx_v, pair_v, rows_v, sel_v, gsem, osem):
        wid = lax.axis_index("s") * NUM_CORES + lax.axis_index("c")
        base = wid * b_per_w
        pltpu.sync_copy(idx_hbm.at[wid], idx_v)

        def gather_start(c, buf):
            for v in range(CHUNK // 16):
                sl = pl.ds(v * 16, 16)
                pair_v[buf, sl] = jax.lax.shift_right_logical(idx_v[c, sl], 1)
            pltpu.async_copy(table_hbm.at[pair_v.at[buf]], rows_v.at[buf], gsem)

        def gather_wait(buf):
            pltpu.make_async_copy(
                table_hbm.at[pair_v.at[buf]], rows_v.at[buf], gsem
            ).wait()

        def out_start(c, buf):
            pltpu.async_copy(
                sel_v.at[buf], out_hbm.at[pl.ds(base + c * CHUNK, CHUNK)], osem
            )

        def out_wait(c, buf):
            pltpu.make_async_copy(
                sel_v.at[buf], out_hbm.at[pl.ds(base + c * CHUNK, CHUNK)], osem
            ).wait()

        gather_start(0, 0)

        def body(c, _):
            buf = lax.rem(c, 2)
            nxt = 1 - buf
            gather_wait(buf)

            @pl.when(c + 1 < n_chunks)
            def _():
                gather_start(c + 1, nxt)

            @pl.when(c >= 2)
            def _():
                out_wait(c - 2, buf)

            for r0 in range(0, CHUNK, 16):
                hv = (idx_v[c, pl.ds(r0, 16)] & 1) * DIM
                offs = [hv[j] for j in range(16)]
                for j0 in range(0, 16, 4):
                    xs = [
                        rows_v[buf, r0 + j0 + jj, pl.ds(offs[j0 + jj] + f0, 16)]
                        for jj in range(4)
                        for f0 in range(0, DIM, 16)
                    ]
                    for k, x in enumerate(xs):
                        jj, f0 = divmod(k, DIM // 16)
                        sel_v[buf, r0 + j0 + jj, pl.ds(f0 * 16, 16)] = x
            out_start(c, buf)
            return 0

        lax.fori_loop(0, n_chunks, body, 0, unroll=False)
        out_wait(n_chunks - 2, (n_chunks - 2) % 2)
        out_wait(n_chunks - 1, (n_chunks - 1) % 2)

    return kern


def kernel(source, W):
    n_total = source.shape[0] * source.shape[1]
    b_per_w = n_total // NUM_WORKERS
    table = W.reshape(500000, 2 * DIM)
    idx = source.reshape(NUM_WORKERS, b_per_w // CHUNK, CHUNK).astype(jnp.int32)
    out = _gather_kernel(n_total)(idx, table)
    return out.reshape(source.shape[0], source.shape[1], DIM)


# speedup vs baseline: 1.6341x; 1.0548x over previous
import functools

import jax
import jax.numpy as jnp
from jax import lax
from jax.experimental import pallas as pl
from jax.experimental.pallas import tpu as pltpu
from jax.experimental.pallas import tpu_sc as plsc

DIM = 64
NUM_CORES = 2
NUM_SUBCORES = 16
NUM_WORKERS = NUM_CORES * NUM_SUBCORES
CHUNK = 128


def _gather_kernel(n_total):
    b_per_w = n_total // NUM_WORKERS
    n_chunks = b_per_w // CHUNK
    mesh = plsc.VectorSubcoreMesh(
        core_axis_name="c",
        subcore_axis_name="s",
        num_cores=NUM_CORES,
        num_subcores=NUM_SUBCORES,
    )

    @functools.partial(
        pl.kernel,
        out_type=jax.ShapeDtypeStruct((n_total, DIM), jnp.float32),
        mesh=mesh,
        scratch_types=[
            pltpu.VMEM((n_chunks, CHUNK), jnp.int32),
            pltpu.VMEM((3, CHUNK), jnp.int32),
            pltpu.VMEM((3, CHUNK, 2 * DIM), jnp.float32),
            pltpu.VMEM((3, CHUNK, DIM), jnp.float32),
            pltpu.SemaphoreType.DMA,
            pltpu.SemaphoreType.DMA,
        ],
        compiler_params=pltpu.CompilerParams(use_tc_tiling_on_sc=True),
    )
    def kern(idx_hbm, table_hbm, out_hbm, idx_v, pair_v, rows_v, sel_v, gsem, osem):
        wid = lax.axis_index("s") * NUM_CORES + lax.axis_index("c")
        base = wid * b_per_w
        pltpu.sync_copy(idx_hbm.at[wid], idx_v)

        def gather_start(c, buf):
            for v in range(CHUNK // 16):
                sl = pl.ds(v * 16, 16)
                pair_v[buf, sl] = jax.lax.shift_right_logical(idx_v[c, sl], 1)
            pltpu.async_copy(table_hbm.at[pair_v.at[buf]], rows_v.at[buf], gsem)

        def gather_wait(buf):
            pltpu.make_async_copy(
                table_hbm.at[pair_v.at[buf]], rows_v.at[buf], gsem
            ).wait()

        def out_start(c, buf):
            pltpu.async_copy(
                sel_v.at[buf], out_hbm.at[pl.ds(base + c * CHUNK, CHUNK)], osem
            )

        def out_wait(c, buf):
            pltpu.make_async_copy(
                sel_v.at[buf], out_hbm.at[pl.ds(base + c * CHUNK, CHUNK)], osem
            ).wait()

        for cc in range(2):
            gather_start(cc, cc)

        def body(c, _):
            buf = lax.rem(c, 3)
            gather_wait(buf)

            @pl.when(c + 2 < n_chunks)
            def _():
                gather_start(c + 2, lax.rem(c + 2, 3))

            @pl.when(c >= 3)
            def _():
                out_wait(c - 3, buf)

            for r0 in range(0, CHUNK, 16):
                hv = (idx_v[c, pl.ds(r0, 16)] & 1) * DIM
                offs = [hv[j] for j in range(16)]
                for j0 in range(0, 16, 4):
                    xs = [
                        rows_v[buf, r0 + j0 + jj, pl.ds(offs[j0 + jj] + f0, 16)]
                        for jj in range(4)
                        for f0 in range(0, DIM, 16)
                    ]
                    for k, x in enumerate(xs):
                        jj, f0 = divmod(k, DIM // 16)
                        sel_v[buf, r0 + j0 + jj, pl.ds(f0 * 16, 16)] = x
            out_start(c, buf)
            return 0

        lax.fori_loop(0, n_chunks, body, 0, unroll=False)
        for cc in range(n_chunks - 3, n_chunks):
            out_wait(cc, cc % 3)

    return kern


def kernel(source, W):
    n_total = source.shape[0] * source.shape[1]
    b_per_w = n_total // NUM_WORKERS
    table = W.reshape(500000, 2 * DIM)
    idx = source.reshape(NUM_WORKERS, b_per_w // CHUNK, CHUNK).astype(jnp.int32)
    out = _gather_kernel(n_total)(idx, table)
    return out.reshape(source.shape[0], source.shape[1], DIM)
